# Initial kernel scaffold; baseline (speedup 1.0000x reference)
import jax
import jax.numpy as jnp
from jax.experimental import pallas as pl


def kernel(input_feature, pos, edge_index, W_neighbor, b_neighbor, W_self, b_self):
    raise NotImplementedError("write your pallas kernel here")


# speedup vs baseline: 2.9587x; 2.9587x over previous
import dataclasses
import functools

import jax
import jax.numpy as jnp
from jax import lax
from jax.experimental import pallas as pl
from jax.experimental.pallas import tpu as pltpu
from jax.experimental.pallas import tpu_sc as plsc

NC = 2
NS = 16
LANES = 16
TILE = 128
HALF = 144
POSC = 112
ONEC = POSC + 3
DISTC = ONEC + 1


def _sc_segment_sums(src, dst, aug0, aug1, px, py, pz, zrow, n_pad, n_tiles):
    mesh = plsc.VectorSubcoreMesh(core_axis_name="c", subcore_axis_name="s")
    cp = pltpu.CompilerParams()
    if "needs_layout_passes" in pltpu.CompilerParams.__dataclass_fields__:
        cp = dataclasses.replace(cp, needs_layout_passes=False)
    if "use_tc_tiling_on_sc" in pltpu.CompilerParams.__dataclass_fields__:
        cp = dataclasses.replace(cp, use_tc_tiling_on_sc=False)
    nblocks = n_pad // TILE

    @functools.partial(
        pl.kernel,
        out_type=[jax.ShapeDtypeStruct((n_pad, HALF), jnp.float32),
                  jax.ShapeDtypeStruct((n_pad, HALF), jnp.float32)],
        mesh=mesh,
        compiler_params=cp,
        scratch_types=[
            pltpu.VMEM((TILE,), jnp.int32),
            pltpu.VMEM((TILE,), jnp.int32),
            pltpu.VMEM((TILE, HALF), jnp.float32),
            pltpu.VMEM((TILE,), jnp.float32),
            pltpu.VMEM((TILE,), jnp.float32),
            pltpu.VMEM((TILE,), jnp.float32),
            pltpu.VMEM((TILE,), jnp.float32),
            pltpu.VMEM((TILE,), jnp.float32),
            pltpu.VMEM((TILE,), jnp.float32),
            pltpu.VMEM_SHARED((n_pad, HALF), jnp.float32),
        ],
    )
    def body(src_hbm, dst_hbm, aug0_hbm, aug1_hbm, px_hbm, py_hbm, pz_hbm,
             zrow_hbm, out0_hbm, out1_hbm,
             srcv, dstv, rows, sxv, syv, szv, dxv, dyv, dzv, acc):
        cid = lax.axis_index("c")
        sid = lax.axis_index("s")

        pltpu.sync_copy(zrow_hbm, rows)

        @pl.loop(sid, nblocks, step=NS)
        def _(b):
            pltpu.sync_copy(rows, acc.at[pl.ds(b * TILE, TILE)])

        plsc.subcore_barrier()

        @pl.loop(sid, n_tiles, step=NS)
        def _(t):
            pltpu.sync_copy(src_hbm.at[pl.ds(t * TILE, TILE)], srcv)
            pltpu.sync_copy(dst_hbm.at[pl.ds(t * TILE, TILE)], dstv)

            @pl.when(cid == 0)
            def _():
                pltpu.sync_copy(aug0_hbm.at[srcv], rows)

            @pl.when(cid == 1)
            def _():
                pltpu.sync_copy(aug1_hbm.at[srcv], rows)
                pltpu.sync_copy(px_hbm.at[srcv], sxv)
                pltpu.sync_copy(py_hbm.at[srcv], syv)
                pltpu.sync_copy(pz_hbm.at[srcv], szv)
                pltpu.sync_copy(px_hbm.at[dstv], dxv)
                pltpu.sync_copy(py_hbm.at[dstv], dyv)
                pltpu.sync_copy(pz_hbm.at[dstv], dzv)

                @pl.loop(0, TILE // LANES)
                def _(i):
                    sl = pl.ds(i * LANES, LANES)
                    dx = dxv[sl] - sxv[sl]
                    dy = dyv[sl] - syv[sl]
                    dz = dzv[sl] - szv[sl]
                    d2 = dx * dx + dy * dy + dz * dz
                    d2c = jnp.maximum(d2, 1e-30)
                    bits = plsc.bitcast(d2c, jnp.int32)
                    y = plsc.bitcast(jnp.int32(0x5F3759DF) - (bits >> 1),
                                     jnp.float32)
                    y = y * (1.5 - 0.5 * d2c * y * y)
                    y = y * (1.5 - 0.5 * d2c * y * y)
                    y = y * (1.5 - 0.5 * d2c * y * y)
                    dist = d2 * y
                    rowid = lax.iota(jnp.int32, LANES) + i * LANES
                    colid = jnp.full((LANES,), DISTC, jnp.int32)
                    plsc.store_scatter(rows, [rowid, colid], dist)

            pltpu.sync_copy(rows, acc.at[dstv], add=True)

        plsc.subcore_barrier()

        @pl.when(cid == 0)
        def _():
            @pl.loop(sid, nblocks, step=NS)
            def _(b):
                pltpu.sync_copy(acc.at[pl.ds(b * TILE, TILE)],
                                out0_hbm.at[pl.ds(b * TILE, TILE)])

        @pl.when(cid == 1)
        def _():
            @pl.loop(sid, nblocks, step=NS)
            def _(b):
                pltpu.sync_copy(acc.at[pl.ds(b * TILE, TILE)],
                                out1_hbm.at[pl.ds(b * TILE, TILE)])

    return body(src, dst, aug0, aug1, px, py, pz, zrow)


def _tc_matmul(a, w, block_m):
    m, k = a.shape
    _, n = w.shape

    def mm(a_ref, w_ref, o_ref):
        o_ref[...] = jnp.dot(a_ref[...], w_ref[...],
                             preferred_element_type=jnp.float32,
                             precision=lax.Precision.HIGHEST)

    return pl.pallas_call(
        mm,
        grid=(m // block_m,),
        in_specs=[pl.BlockSpec((block_m, k), lambda i: (i, 0)),
                  pl.BlockSpec((k, n), lambda i: (0, 0))],
        out_specs=pl.BlockSpec((block_m, n), lambda i: (i, 0)),
        out_shape=jax.ShapeDtypeStruct((m, n), jnp.float32),
    )(a, w)


def kernel(input_feature, pos, edge_index, W_neighbor, b_neighbor, W_self,
           b_self):
    n, d_in = input_feature.shape
    e = edge_index.shape[1]
    d_out = W_self.shape[1]
    assert e % TILE == 0
    n_tiles = e // TILE
    n_pad = ((n + TILE - 1) // TILE) * TILE

    feat = input_feature.astype(jnp.float32)
    pos = pos.astype(jnp.float32)
    src = edge_index[0].astype(jnp.int32)
    dst = edge_index[1].astype(jnp.int32)

    aug0 = feat[:, :HALF]
    aug1 = jnp.concatenate(
        [feat[:, HALF:], pos, jnp.ones((n, 1), jnp.float32),
         jnp.zeros((n, HALF - DISTC), jnp.float32)], axis=1)
    px = pos[:, 0] + 0.0
    py = pos[:, 1] + 0.0
    pz = pos[:, 2] + 0.0
    zrow = jnp.zeros((TILE, HALF), jnp.float32)

    acc0, acc1 = _sc_segment_sums(src, dst, aug0, aug1, px, py, pz, zrow,
                                  n_pad, n_tiles)

    feat_sum = jnp.concatenate([acc0[:n], acc1[:n, :POSC]], axis=1)
    possum = acc1[:n, POSC:POSC + 3]
    deg = acc1[:n, ONEC:ONEC + 1]
    distsum = acc1[:n, DISTC:DISTC + 1]

    a = jnp.concatenate(
        [feat_sum, deg * feat, deg * pos - possum, distsum, deg,
         jnp.zeros((n, 3), jnp.float32)], axis=1)
    block_m = 1280
    m_pad = ((n + block_m - 1) // block_m) * block_m
    a = jnp.pad(a, ((0, m_pad - n), (0, 0)))
    w_big = jnp.concatenate(
        [W_neighbor[:d_in], W_self, W_neighbor[d_in:d_in + 3],
         W_neighbor[d_in + 3:d_in + 4], (b_neighbor + b_self)[None],
         jnp.zeros((3, d_out), jnp.float32)], axis=0)

    out = _tc_matmul(a, w_big, block_m=block_m)
    return out[:n]


# speedup vs baseline: 6.4106x; 2.1667x over previous
import dataclasses
import functools

import jax
import jax.numpy as jnp
from jax import lax
from jax.experimental import pallas as pl
from jax.experimental.pallas import tpu as pltpu
from jax.experimental.pallas import tpu_sc as plsc

NC = 2
NS = 16
LANES = 16
TILE = 128
HALF = 144
POSC = 112
ONEC = POSC + 3
DISTC = ONEC + 1


def _sc_segment_sums(src, dst, aug0, aug1, px, py, pz, zrow, n_pad, n_tiles):
    mesh = plsc.VectorSubcoreMesh(core_axis_name="c", subcore_axis_name="s")
    cp = pltpu.CompilerParams()
    if "needs_layout_passes" in pltpu.CompilerParams.__dataclass_fields__:
        cp = dataclasses.replace(cp, needs_layout_passes=False)
    if "use_tc_tiling_on_sc" in pltpu.CompilerParams.__dataclass_fields__:
        cp = dataclasses.replace(cp, use_tc_tiling_on_sc=False)
    nblocks = n_pad // TILE

    bufset = [
        pltpu.VMEM((TILE,), jnp.int32),
        pltpu.VMEM((TILE,), jnp.int32),
        pltpu.VMEM((TILE, HALF), jnp.float32),
        pltpu.VMEM((TILE,), jnp.float32),
        pltpu.VMEM((TILE,), jnp.float32),
        pltpu.VMEM((TILE,), jnp.float32),
        pltpu.VMEM((TILE,), jnp.float32),
        pltpu.VMEM((TILE,), jnp.float32),
        pltpu.VMEM((TILE,), jnp.float32),
        pltpu.SemaphoreType.DMA,
        pltpu.SemaphoreType.DMA,
    ]

    @functools.partial(
        pl.kernel,
        out_type=[jax.ShapeDtypeStruct((n_pad, HALF), jnp.float32),
                  jax.ShapeDtypeStruct((n_pad, HALF), jnp.float32)],
        mesh=mesh,
        compiler_params=cp,
        scratch_types=bufset + bufset + [
            pltpu.VMEM_SHARED((n_pad, HALF), jnp.float32),
        ],
    )
    def body(src_hbm, dst_hbm, aug0_hbm, aug1_hbm, px_hbm, py_hbm, pz_hbm,
             zrow_hbm, out0_hbm, out1_hbm, *refs):
        bufs = (refs[0:11], refs[11:22])
        acc = refs[22]
        cid = lax.axis_index("c")
        sid = lax.axis_index("s")
        nk = (n_tiles - sid + NS - 1) // NS

        def idx_copies(k, buf):
            t = (sid + k * NS) * TILE
            srcv, dstv = buf[0], buf[1]
            return [pltpu.make_async_copy(src_hbm.at[pl.ds(t, TILE)], srcv,
                                          buf[10]),
                    pltpu.make_async_copy(dst_hbm.at[pl.ds(t, TILE)], dstv,
                                          buf[10])]

        def gather_copies(buf):
            srcv, dstv, rows = buf[0], buf[1], buf[2]
            if_sem = buf[9]
            g0 = [pltpu.make_async_copy(aug0_hbm.at[srcv], rows, if_sem)]
            g1 = [pltpu.make_async_copy(aug1_hbm.at[srcv], rows, if_sem),
                  pltpu.make_async_copy(px_hbm.at[srcv], buf[3], if_sem),
                  pltpu.make_async_copy(py_hbm.at[srcv], buf[4], if_sem),
                  pltpu.make_async_copy(pz_hbm.at[srcv], buf[5], if_sem),
                  pltpu.make_async_copy(px_hbm.at[dstv], buf[6], if_sem),
                  pltpu.make_async_copy(py_hbm.at[dstv], buf[7], if_sem),
                  pltpu.make_async_copy(pz_hbm.at[dstv], buf[8], if_sem)]
            return g0, g1

        def start_gathers(buf):
            g0, g1 = gather_copies(buf)

            @pl.when(cid == 0)
            def _():
                for c in g0:
                    c.start()

            @pl.when(cid == 1)
            def _():
                for c in g1:
                    c.start()

        def wait_gathers(buf):
            g0, g1 = gather_copies(buf)

            @pl.when(cid == 0)
            def _():
                for c in g0:
                    c.wait()

            @pl.when(cid == 1)
            def _():
                for c in g1:
                    c.wait()

        def compute_dist(buf):
            rows = buf[2]

            @pl.when(cid == 1)
            def _():
                @pl.loop(0, TILE // LANES)
                def _(i):
                    sl = pl.ds(i * LANES, LANES)
                    dx = buf[6][sl] - buf[3][sl]
                    dy = buf[7][sl] - buf[4][sl]
                    dz = buf[8][sl] - buf[5][sl]
                    d2 = dx * dx + dy * dy + dz * dz
                    d2c = jnp.maximum(d2, 1e-30)
                    bits = plsc.bitcast(d2c, jnp.int32)
                    y = plsc.bitcast(jnp.int32(0x5F3759DF) - (bits >> 1),
                                     jnp.float32)
                    y = y * (1.5 - 0.5 * d2c * y * y)
                    y = y * (1.5 - 0.5 * d2c * y * y)
                    y = y * (1.5 - 0.5 * d2c * y * y)
                    dist = d2 * y
                    rowid = lax.iota(jnp.int32, LANES) + i * LANES
                    colid = jnp.full((LANES,), DISTC, jnp.int32)
                    plsc.store_scatter(rows, [rowid, colid], dist)

        def process(k, cur, nxt):
            @pl.when(k + 1 < nk)
            def _():
                for c in idx_copies(k + 1, nxt):
                    c.wait()
                start_gathers(nxt)
            wait_gathers(cur)
            compute_dist(cur)
            pltpu.sync_copy(cur[2], acc.at[cur[1]], add=True)

            @pl.when(k + 2 < nk)
            def _():
                for c in idx_copies(k + 2, cur):
                    c.start()

        pltpu.sync_copy(zrow_hbm, bufs[0][2])

        @pl.loop(sid, nblocks, step=NS)
        def _(b):
            pltpu.sync_copy(bufs[0][2], acc.at[pl.ds(b * TILE, TILE)])

        plsc.subcore_barrier()

        @pl.when(nk > 0)
        def _():
            for c in idx_copies(0, bufs[0]):
                c.start()
                c.wait()
            start_gathers(bufs[0])

            @pl.when(1 < nk)
            def _():
                for c in idx_copies(1, bufs[1]):
                    c.start()

        @pl.loop(0, (nk + 1) // 2)
        def _(p):
            process(2 * p, bufs[0], bufs[1])

            @pl.when(2 * p + 1 < nk)
            def _():
                process(2 * p + 1, bufs[1], bufs[0])

        plsc.subcore_barrier()

        @pl.when(cid == 0)
        def _():
            @pl.loop(sid, nblocks, step=NS)
            def _(b):
                pltpu.sync_copy(acc.at[pl.ds(b * TILE, TILE)],
                                out0_hbm.at[pl.ds(b * TILE, TILE)])

        @pl.when(cid == 1)
        def _():
            @pl.loop(sid, nblocks, step=NS)
            def _(b):
                pltpu.sync_copy(acc.at[pl.ds(b * TILE, TILE)],
                                out1_hbm.at[pl.ds(b * TILE, TILE)])

    return body(src, dst, aug0, aug1, px, py, pz, zrow)


def _tc_matmul(a, w, block_m):
    m, k = a.shape
    _, n = w.shape

    def mm(a_ref, w_ref, o_ref):
        o_ref[...] = jnp.dot(a_ref[...], w_ref[...],
                             preferred_element_type=jnp.float32,
                             precision=lax.Precision.HIGHEST)

    return pl.pallas_call(
        mm,
        grid=(m // block_m,),
        in_specs=[pl.BlockSpec((block_m, k), lambda i: (i, 0)),
                  pl.BlockSpec((k, n), lambda i: (0, 0))],
        out_specs=pl.BlockSpec((block_m, n), lambda i: (i, 0)),
        out_shape=jax.ShapeDtypeStruct((m, n), jnp.float32),
    )(a, w)


def kernel(input_feature, pos, edge_index, W_neighbor, b_neighbor, W_self,
           b_self):
    n, d_in = input_feature.shape
    e = edge_index.shape[1]
    d_out = W_self.shape[1]
    assert e % TILE == 0
    n_tiles = e // TILE
    n_pad = ((n + TILE - 1) // TILE) * TILE

    feat = input_feature.astype(jnp.float32)
    pos = pos.astype(jnp.float32)
    src = edge_index[0].astype(jnp.int32)
    dst = edge_index[1].astype(jnp.int32)

    aug0 = feat[:, :HALF]
    aug1 = jnp.concatenate(
        [feat[:, HALF:], pos, jnp.ones((n, 1), jnp.float32),
         jnp.zeros((n, HALF - DISTC), jnp.float32)], axis=1)
    px = pos[:, 0] + 0.0
    py = pos[:, 1] + 0.0
    pz = pos[:, 2] + 0.0
    zrow = jnp.zeros((TILE, HALF), jnp.float32)

    acc0, acc1 = _sc_segment_sums(src, dst, aug0, aug1, px, py, pz, zrow,
                                  n_pad, n_tiles)

    feat_sum = jnp.concatenate([acc0[:n], acc1[:n, :POSC]], axis=1)
    possum = acc1[:n, POSC:POSC + 3]
    deg = acc1[:n, ONEC:ONEC + 1]
    distsum = acc1[:n, DISTC:DISTC + 1]

    a = jnp.concatenate(
        [feat_sum, deg * feat, deg * pos - possum, distsum, deg,
         jnp.zeros((n, 3), jnp.float32)], axis=1)
    block_m = 1280
    m_pad = ((n + block_m - 1) // block_m) * block_m
    a = jnp.pad(a, ((0, m_pad - n), (0, 0)))
    w_big = jnp.concatenate(
        [W_neighbor[:d_in], W_self, W_neighbor[d_in:d_in + 3],
         W_neighbor[d_in + 3:d_in + 4], (b_neighbor + b_self)[None],
         jnp.zeros((3, d_out), jnp.float32)], axis=0)

    out = _tc_matmul(a, w_big, block_m=block_m)
    return out[:n]
